# baseline (speedup 1.0000x reference)
import jax
import jax.numpy as jnp
from jax.experimental import pallas as pl
from jax.experimental.pallas import tpu as pltpu

_NUM_HEADS = 8
_DIM = 32
_K = 1024
_TB = 4096


def _vq_body(x_ref, mt_ref, out_ref, msq_ref):
    @pl.when(pl.program_id(0) == 0)
    def _init_msq():
        for h in range(_NUM_HEADS):
            mt = mt_ref[h * _DIM:(h + 1) * _DIM, :]
            msq_ref[h:h + 1, :] = 0.5 * jnp.sum(mt * mt, axis=0, keepdims=True)

    iota_f = jax.lax.broadcasted_iota(jnp.int32, (_TB, _K), 1).astype(jnp.float32)
    ids = []
    for h in range(_NUM_HEADS):
        xh = x_ref[:, h * _DIM:(h + 1) * _DIM]
        mt = mt_ref[h * _DIM:(h + 1) * _DIM, :]
        cross = jnp.dot(xh, mt, preferred_element_type=jnp.float32)
        dists = msq_ref[h:h + 1, :] - cross
        vmin = jnp.min(dists, axis=1, keepdims=True)
        idxf = jnp.min(jnp.where(dists == vmin, iota_f, float(_K)), axis=1)
        ids.append(idxf.astype(jnp.int32))
    out_ref[...] = jnp.stack(ids, axis=1)


def kernel(x, means):
    b, n, feat = x.shape
    x2 = x.reshape(b * n, feat)
    mt = means.transpose(0, 2, 1).reshape(_NUM_HEADS * _DIM, _K)
    grid = ((b * n) // _TB,)
    out = pl.pallas_call(
        _vq_body,
        grid=grid,
        in_specs=[
            pl.BlockSpec((_TB, feat), lambda i: (i, 0)),
            pl.BlockSpec((_NUM_HEADS * _DIM, _K), lambda i: (0, 0)),
        ],
        out_specs=pl.BlockSpec((_TB, _NUM_HEADS), lambda i: (i, 0)),
        out_shape=jax.ShapeDtypeStruct((b * n, _NUM_HEADS), jnp.int32),
        scratch_shapes=[pltpu.VMEM((_NUM_HEADS, _K), jnp.float32)],
    )(x2, mt)
    return out.reshape(b, n, _NUM_HEADS)


# speedup vs baseline: 1.0080x; 1.0080x over previous
import jax
import jax.numpy as jnp
from jax.experimental import pallas as pl
from jax.experimental.pallas import tpu as pltpu

_NUM_HEADS = 8
_DIM = 32
_K = 1024
_TB = 2048


def _vq_body(x_ref, mt_ref, out_ref, msq_ref):
    @pl.when(pl.program_id(0) == 0)
    def _init_msq():
        for h in range(_NUM_HEADS):
            mt = mt_ref[h * _DIM:(h + 1) * _DIM, :]
            msq_ref[h:h + 1, :] = 0.5 * jnp.sum(mt * mt, axis=0, keepdims=True)

    iota_f = jax.lax.broadcasted_iota(jnp.int32, (_TB, _K), 1).astype(jnp.float32)
    ids = []
    for h in range(_NUM_HEADS):
        xh = x_ref[:, h * _DIM:(h + 1) * _DIM]
        mt = mt_ref[h * _DIM:(h + 1) * _DIM, :]
        cross = jnp.dot(xh, mt, preferred_element_type=jnp.float32)
        dists = msq_ref[h:h + 1, :] - cross
        vmin = jnp.min(dists, axis=1, keepdims=True)
        idxf = jnp.min(jnp.where(dists == vmin, iota_f, float(_K)), axis=1)
        ids.append(idxf.astype(jnp.int32))
    out_ref[...] = jnp.stack(ids, axis=1)


def kernel(x, means):
    b, n, feat = x.shape
    x2 = x.reshape(b * n, feat)
    mt = means.transpose(0, 2, 1).reshape(_NUM_HEADS * _DIM, _K)
    grid = ((b * n) // _TB,)
    out = pl.pallas_call(
        _vq_body,
        grid=grid,
        in_specs=[
            pl.BlockSpec((_TB, feat), lambda i: (i, 0)),
            pl.BlockSpec((_NUM_HEADS * _DIM, _K), lambda i: (0, 0)),
        ],
        out_specs=pl.BlockSpec((_TB, _NUM_HEADS), lambda i: (i, 0)),
        out_shape=jax.ShapeDtypeStruct((b * n, _NUM_HEADS), jnp.int32),
        scratch_shapes=[pltpu.VMEM((_NUM_HEADS, _K), jnp.float32)],
    )(x2, mt)
    return out.reshape(b, n, _NUM_HEADS)
